# baseline (speedup 1.0000x reference)
import jax
import jax.numpy as jnp
from jax import lax
from jax.experimental import pallas as pl
from jax.experimental.pallas import tpu as pltpu
from jax.experimental.pallas import tpu_sc as plsc

ALPHA = 0.3
BETA = 0.7
GAMMA = 1.33
SMOOTH = 1e-06
KEEP_RATIO = 0.3
LOVASZ_WEIGHT = 0.2

B = 16
TOTAL = 262144
K_ALL = max(1, int(TOTAL * KEEP_RATIO))
NBINS = 1024
BIN_OFF = (127 - 20) << 5
NEG_BIG = -3.0e38

CHUNKS_A = 8
ROWS_A = 64
LANES_A = 4096
NTEC = 32
PER_TEC = TOTAL // 2
SC_CHUNK = 2048
TBL_WORDS = 16 * NBINS


def _tca_body(x_ref, tgt_ref, tis_ref, okey_ref, lkey_ref, sc_ref, acc_ref):
    c = pl.program_id(1)

    x = x_ref[0]
    tgt = tgt_ref[0]
    tis = tis_ref[0]
    t = tgt.astype(jnp.float32)
    tisf = tis.astype(jnp.float32)

    ax = jnp.abs(x)
    bce = jnp.maximum(x, 0.0) - x * t + jnp.log(1.0 + jnp.exp(-ax))
    posm = (tgt == 1) & (tis == 1)
    negm = (tgt == 0) & (tis == 1)

    okey_ref[0] = jnp.where(negm, bce, 0.0)

    sig = 1.0 / (1.0 + jnp.exp(-x))
    e = 1.0 - x * (2.0 * t - 1.0)
    r = jnp.where(e > 0.0, e, 0.0)
    rb = lax.bitcast_convert_type(r, jnp.int32)
    rb = jnp.where(r > 0.0, (rb & ~1) | tgt, 0)
    lkey_ref[0] = lax.bitcast_convert_type(rb, jnp.float32)

    ri = lax.broadcasted_iota(jnp.int32, (ROWS_A // CHUNKS_A, LANES_A), 0)
    ci = lax.broadcasted_iota(jnp.int32, (ROWS_A // CHUNKS_A, LANES_A), 1)
    first = (ri == 0) & (ci == 0)

    pos_sum = jnp.sum(jnp.where(posm, bce, 0.0))
    n_pos = jnp.sum(jnp.where(posm, 1.0, 0.0))
    n_neg = jnp.sum(jnp.where(negm, 1.0, 0.0))
    tis_cnt = jnp.sum(tisf)
    tis_max = jnp.max(jnp.where(tis == 1, bce, NEG_BIG))
    bce0 = jnp.where(c == 0, jnp.sum(jnp.where(first, bce * tisf, 0.0)), NEG_BIG)
    tp = jnp.sum(sig * t)
    fn = jnp.sum((1.0 - sig) * t)
    fp = jnp.sum(sig * (1.0 - t))
    p_sum = jnp.sum(t)

    li = lax.broadcasted_iota(jnp.int32, (1, 128), 1)
    upd = jnp.zeros((1, 128), jnp.float32)
    for k, v in ((0, pos_sum), (1, n_pos), (2, n_neg), (3, tis_cnt),
                 (6, tp), (7, fn), (8, fp), (9, p_sum)):
        upd = upd + jnp.where(li == k, v, 0.0)
    mx = jnp.where(li == 4, tis_max, NEG_BIG) + jnp.where(li == 5, bce0 - NEG_BIG, 0.0)
    ismax = (li == 4) | (li == 5)

    @pl.when(c == 0)
    def _():
        acc_ref[...] = jnp.where(ismax, mx, upd)

    @pl.when(c > 0)
    def _():
        prev = acc_ref[...]
        acc_ref[...] = jnp.where(ismax, jnp.maximum(prev, mx), prev + upd)

    @pl.when(c == CHUNKS_A - 1)
    def _():
        sc_ref[0] = acc_ref[...]


def _run_tca(x, tgt, tis):
    bs_in = pl.BlockSpec((1, ROWS_A // CHUNKS_A, LANES_A), lambda s, c: (s, c, 0))
    bs_sc = pl.BlockSpec((1, 1, 128), lambda s, c: (s, 0, 0))
    return pl.pallas_call(
        _tca_body,
        grid=(B, CHUNKS_A),
        in_specs=[bs_in, bs_in, bs_in],
        out_specs=[bs_in, bs_in, bs_sc],
        out_shape=[
            jax.ShapeDtypeStruct((B, ROWS_A, LANES_A), jnp.float32),
            jax.ShapeDtypeStruct((B, ROWS_A, LANES_A), jnp.float32),
            jax.ShapeDtypeStruct((B, 1, 128), jnp.float32),
        ],
        scratch_shapes=[pltpu.VMEM((1, 128), jnp.float32)],
    )(x, tgt, tis)


def _bin16(v):
    bits = lax.bitcast_convert_type(v, jnp.int32)
    bn = jnp.clip((bits >> 18) - BIN_OFF, 0, NBINS - 1)
    return bits, bn


NCH = PER_TEC // SC_CHUNK
UNROLL = 4


def _dbuf_stream(src, base, buf2, sem0, sem1, per_vreg, init):
    def desc(slot, k, sem):
        return pltpu.make_async_copy(
            src.at[pl.ds(base + k * SC_CHUNK, SC_CHUNK)],
            buf2.at[pl.ds(slot * SC_CHUNK, SC_CHUNK)],
            sem,
        )

    desc(0, 0, sem0).start()

    def chunk(k, carry):
        @pl.when((k + 1 < NCH) & (k % 2 == 0))
        def _():
            desc(1, k + 1, sem1).start()

        @pl.when((k + 1 < NCH) & (k % 2 == 1))
        def _():
            desc(0, k + 1, sem0).start()

        @pl.when(k % 2 == 0)
        def _():
            desc(0, k, sem0).wait()

        @pl.when(k % 2 == 1)
        def _():
            desc(1, k, sem1).wait()

        off = (k % 2) * SC_CHUNK

        def inner(j, c2):
            for u in range(UNROLL):
                c2 = per_vreg(buf2[pl.ds(off + (j * UNROLL + u) * 16, 16)], c2, u)
            return c2

        return lax.fori_loop(0, SC_CHUNK // (16 * UNROLL), inner, carry)

    return lax.fori_loop(0, NCH, chunk, init)


def _scb_body(okey, lkey, ocnt, osum, lpos, lneg,
              buf2, t0, t1, t2, t3, idxv, zbuf, s0m, s1m, s2m, s3m, sem0, sem1):
    core = lax.axis_index("c")
    sub = lax.axis_index("s")
    wid = sub * 2 + core
    base = wid * PER_TEC
    lane = lax.iota(jnp.int32, 16)
    ones = jnp.ones((16,), jnp.float32)
    zeros = jnp.zeros((16,), jnp.float32)

    def bidx(i, carry):
        idxv[pl.ds(i * 16, 16)] = jnp.zeros((16,), jnp.int32) + (sub * NBINS + i)
        return carry

    lax.fori_loop(0, TBL_WORDS // 16, bidx, 0)

    def zv(i, carry):
        zbuf[pl.ds(i * 16, 16)] = zeros
        return carry

    lax.fori_loop(0, NBINS // 16, zv, 0)
    row = pl.ds(sub * NBINS, NBINS)
    for spm in (s0m, s1m, s2m, s3m):
        pltpu.sync_copy(zbuf, spm.at[row])

    def zero_tables():
        def z(i, carry):
            for u in range(8):
                t0[pl.ds((i * 8 + u) * 16, 16)] = zeros
                t1[pl.ds((i * 8 + u) * 16, 16)] = zeros
                t2[pl.ds((i * 8 + u) * 16, 16)] = zeros
                t3[pl.ds((i * 8 + u) * 16, 16)] = zeros
            return carry

        lax.fori_loop(0, TBL_WORDS // 128, z, 0)

    def ohem_vreg(v, carry, u):
        ta, tb = (t0, t1) if u % 2 == 0 else (t2, t3)
        bits, bn = _bin16(v)
        idx = bn * 16 + lane
        plsc.addupdate_scatter(ta, [idx], ones)
        plsc.addupdate_scatter(tb, [idx], v)
        return carry

    def lov_vreg(v, carry, u):
        ta, tb = (t0, t1) if u % 2 == 0 else (t2, t3)
        bits, bn = _bin16(v)
        idx = bn * 16 + lane
        gf = (bits & 1).astype(jnp.float32)
        plsc.addupdate_scatter(ta, [idx], gf)
        plsc.addupdate_scatter(tb, [idx], jnp.where((v > 0.0) & (gf == 0.0), 1.0, 0.0))
        return carry

    zero_tables()
    _dbuf_stream(okey, base, buf2, sem0, sem1, ohem_vreg, 0)
    pltpu.sync_copy(t0, s0m.at[idxv], add=True)
    pltpu.sync_copy(t2, s0m.at[idxv], add=True)
    pltpu.sync_copy(t1, s1m.at[idxv], add=True)
    pltpu.sync_copy(t3, s1m.at[idxv], add=True)

    zero_tables()
    _dbuf_stream(lkey, base, buf2, sem0, sem1, lov_vreg, 0)
    pltpu.sync_copy(t0, s2m.at[idxv], add=True)
    pltpu.sync_copy(t2, s2m.at[idxv], add=True)
    pltpu.sync_copy(t1, s3m.at[idxv], add=True)
    pltpu.sync_copy(t3, s3m.at[idxv], add=True)

    out_off = pl.ds((core * 16 + sub) * NBINS, NBINS)
    pltpu.sync_copy(s0m.at[row], ocnt.at[out_off])
    pltpu.sync_copy(s1m.at[row], osum.at[out_off])
    pltpu.sync_copy(s2m.at[row], lpos.at[out_off])
    pltpu.sync_copy(s3m.at[row], lneg.at[out_off])


def _run_scb(okey_flat, lkey_flat):
    mesh = plsc.VectorSubcoreMesh(core_axis_name="c", subcore_axis_name="s")
    f = pl.kernel(
        _scb_body,
        mesh=mesh,
        compiler_params=pltpu.CompilerParams(needs_layout_passes=False),
        out_type=[jax.ShapeDtypeStruct((2 * B * NBINS,), jnp.float32)] * 4,
        scratch_types=[
            pltpu.VMEM((2 * SC_CHUNK,), jnp.float32),
            pltpu.VMEM((TBL_WORDS,), jnp.float32),
            pltpu.VMEM((TBL_WORDS,), jnp.float32),
            pltpu.VMEM((TBL_WORDS,), jnp.float32),
            pltpu.VMEM((TBL_WORDS,), jnp.float32),
            pltpu.VMEM((TBL_WORDS,), jnp.int32),
            pltpu.VMEM((NBINS,), jnp.float32),
            pltpu.VMEM_SHARED((B * NBINS,), jnp.float32),
            pltpu.VMEM_SHARED((B * NBINS,), jnp.float32),
            pltpu.VMEM_SHARED((B * NBINS,), jnp.float32),
            pltpu.VMEM_SHARED((B * NBINS,), jnp.float32),
            pltpu.SemaphoreType.DMA,
            pltpu.SemaphoreType.DMA,
        ],
    )
    return f(okey_flat, lkey_flat)


def _suffix_excl(X):
    ut = (lax.broadcasted_iota(jnp.int32, (128, 128), 0)
          > lax.broadcasted_iota(jnp.int32, (128, 128), 1)).astype(jnp.float32)
    within = jnp.dot(X, ut, preferred_element_type=jnp.float32)
    rows = jnp.sum(X, axis=1, keepdims=True)
    m8 = (lax.broadcasted_iota(jnp.int32, (8, 8), 1)
          > lax.broadcasted_iota(jnp.int32, (8, 8), 0)).astype(jnp.float32)
    above = jnp.dot(m8, rows, preferred_element_type=jnp.float32)
    return within + above


def _tcc_body(ocnt_ref, osum_ref, lpos_ref, lneg_ref, sc_ref, w1_ref, w2_ref, sc2_ref):
    red = lambda ref: jnp.sum(ref[...], axis=(0, 1))
    cnt = red(ocnt_ref)
    sm = red(osum_ref)
    lp = red(lpos_ref)
    ln = red(lneg_ref)

    srow = sc_ref[0]
    li = lax.broadcasted_iota(jnp.int32, (1, 128), 1)
    g = lambda k: jnp.sum(jnp.where(li == k, srow, 0.0))
    pos_sum, n_pos, num_neg = g(0), g(1), g(2)
    tis_cnt, tis_max, bce0 = g(3), g(4), g(5)
    tp, fn, fp, p = g(6), g(7), g(8), g(9)

    bi = lax.broadcasted_iota(jnp.int32, (8, 128), 0) * 128 + \
        lax.broadcasted_iota(jnp.int32, (8, 128), 1)
    is0 = (bi == 0).astype(jnp.float32)

    cnt = cnt - is0 * (jnp.float32(TOTAL) - num_neg)
    S = _suffix_excl(cnt)
    Ssum = _suffix_excl(sm)
    m = jnp.minimum(jnp.maximum(0.0, jnp.float32(K_ALL) - n_pos), num_neg)
    T = S + cnt
    mask = (S < m) & (m <= T)
    pick = lambda A: jnp.sum(jnp.where(mask, A, 0.0))
    S_t, cnt_t, sum_t, above_sum = pick(S), pick(cnt), pick(sm), pick(Ssum)
    lo_g = lax.bitcast_convert_type((bi + BIN_OFF) << 18, jnp.float32)
    hi_g = lax.bitcast_convert_type((bi + 1 + BIN_OFF) << 18, jnp.float32)
    lo, hi = pick(lo_g), pick(hi_g)
    mu = sum_t / jnp.maximum(cnt_t, 1e-30)
    h = jnp.maximum(0.0, jnp.minimum(hi - mu, mu - lo))
    kprime = m - S_t
    phi = kprime / jnp.maximum(cnt_t, 1e-30)
    neg_sum = above_sum + kprime * (mu + h * (1.0 - phi))
    kept = n_pos + m
    kept_loss = (pos_sum + neg_sum) / kept
    empty_loss = jnp.where(tis_cnt > 0, tis_max, bce0)
    ohem_i = jnp.where(kept == 0, empty_loss, kept_loss)

    tv = (tp + SMOOTH) / (tp + ALPHA * fn + BETA * fp + SMOOTH)
    omt = 1.0 - tv
    ft_i = jnp.where(omt > 0, jnp.exp(GAMMA * jnp.log(jnp.maximum(omt, 1e-38))), 0.0)

    CB = _suffix_excl(lp)
    NB = _suffix_excl(ln)
    n_neg_c = jnp.float32(TOTAL) - p
    a1 = p + NB + 0.5 * ln
    a2 = p + NB + 0.5 * (ln - 1.0)
    w1n = 1.0 / a1
    w2n = (p - CB - 0.5 * lp) / (a2 * (a2 + 1.0))
    w1a = (CB + 0.5 * (lp + 1.0)) / jnp.maximum(p, 1e-30)
    allpos = n_neg_c == 0
    w1_ref[0] = jnp.where(allpos, w1a, w1n)
    w2_ref[0] = jnp.where(allpos, 0.0, w2n)

    posb = jnp.where(p > 0, 1.0, 0.0)
    sc2_ref[0] = jnp.where(li == 0, ohem_i, 0.0) + \
        jnp.where(li == 1, ft_i, 0.0) + jnp.where(li == 2, posb, 0.0)


def _run_tcc(hists, scal):
    bs_h = pl.BlockSpec((2, 1, 8, 128), lambda s: (0, s, 0, 0))
    bs_s = pl.BlockSpec((1, 1, 128), lambda s: (s, 0, 0))
    bs_w = pl.BlockSpec((1, 8, 128), lambda s: (s, 0, 0))
    return pl.pallas_call(
        _tcc_body,
        grid=(B,),
        in_specs=[bs_h, bs_h, bs_h, bs_h, bs_s],
        out_specs=[bs_w, bs_w, bs_s],
        out_shape=[
            jax.ShapeDtypeStruct((B, 8, 128), jnp.float32),
            jax.ShapeDtypeStruct((B, 8, 128), jnp.float32),
            jax.ShapeDtypeStruct((B, 1, 128), jnp.float32),
        ],
    )(*hists, scal)


def _scd_body(w1f, w2f, lkey, lout, w1v, w2v, w1r, w2r, buf2, accv, sem0, sem1):
    wid = lax.axis_index("s") * 2 + lax.axis_index("c")
    s = wid // 2
    base = wid * PER_TEC
    lane = lax.iota(jnp.int32, 16)
    pltpu.sync_copy(w1f.at[pl.ds(s * NBINS, NBINS)], w1v)
    pltpu.sync_copy(w2f.at[pl.ds(s * NBINS, NBINS)], w2v)

    def rep(i, carry):
        a = w1v[pl.ds(i * 16, 16)]
        b = w2v[pl.ds(i * 16, 16)]
        bins16 = (jnp.zeros((16,), jnp.int32) + i * 16 + lane) * 16
        for l in range(16):
            plsc.store_scatter(w1r, [bins16 + l], a)
            plsc.store_scatter(w2r, [bins16 + l], b)
        return carry

    lax.fori_loop(0, NBINS // 16, rep, 0)

    def lov_vreg(v, acc, u):
        bits, bn = _bin16(v)
        idx = bn * 16 + lane
        wa = plsc.load_gather(w1r, [idx])
        wb = plsc.load_gather(w2r, [idx])
        w = jnp.where((bits & 1) == 1, wa, wb)
        return acc + v * w

    acc = _dbuf_stream(lkey, base, buf2, sem0, sem1, lov_vreg,
                       jnp.zeros((16,), jnp.float32))
    accv[pl.ds(0, 16)] = acc
    pltpu.sync_copy(accv, lout.at[pl.ds(wid * 16, 16)])


def _run_scd(w1, w2, lkey_flat):
    mesh = plsc.VectorSubcoreMesh(core_axis_name="c", subcore_axis_name="s")
    f = pl.kernel(
        _scd_body,
        mesh=mesh,
        compiler_params=pltpu.CompilerParams(needs_layout_passes=False),
        out_type=[jax.ShapeDtypeStruct((NTEC * 16,), jnp.float32)],
        scratch_types=[
            pltpu.VMEM((NBINS,), jnp.float32),
            pltpu.VMEM((NBINS,), jnp.float32),
            pltpu.VMEM((TBL_WORDS,), jnp.float32),
            pltpu.VMEM((TBL_WORDS,), jnp.float32),
            pltpu.VMEM((2 * SC_CHUNK,), jnp.float32),
            pltpu.VMEM((16,), jnp.float32),
            pltpu.SemaphoreType.DMA,
            pltpu.SemaphoreType.DMA,
        ],
    )
    return f(w1, w2, lkey_flat)


def _tce_body(sc2_ref, lov_ref, out_ref):
    sc2 = sc2_ref[...]
    li = lax.broadcasted_iota(jnp.int32, (B, 1, 128), 2)
    col = lambda k: jnp.sum(jnp.where(li == k, sc2, 0.0), axis=(1, 2))
    ohem_i, ft_i, posb = col(0), col(1), col(2)
    lov_i = jnp.sum(lov_ref[...], axis=1)
    n_pos_b = jnp.sum(posb)
    ohem_term = jnp.sum(ohem_i) / jnp.float32(B)
    ft_term = jnp.sum(jnp.where(posb > 0, ft_i, 0.0)) / n_pos_b
    lov_term = jnp.sum(jnp.where(posb > 0, lov_i, 0.0)) / n_pos_b
    full = ohem_term + ft_term + LOVASZ_WEIGHT * lov_term
    out_ref[...] = jnp.broadcast_to(jnp.where(n_pos_b > 0, full, ohem_term), (1, 1))


def _run_tce(sc2, lovpart):
    return pl.pallas_call(
        _tce_body,
        out_shape=jax.ShapeDtypeStruct((1, 1), jnp.float32),
    )(sc2, lovpart.reshape(B, NTEC * 16 // B))


def kernel(logits, targets, tissue_mask):
    x = logits.reshape(B, ROWS_A, LANES_A)
    tgt = targets.reshape(B, ROWS_A, LANES_A)
    tis = tissue_mask.reshape(B, ROWS_A, LANES_A)

    okey, lkey, scal = _run_tca(x, tgt, tis)
    okey_f = okey.reshape(-1)
    lkey_f = lkey.reshape(-1)

    hists = _run_scb(okey_f, lkey_f)
    hists4 = [h.reshape(2, B, 8, 128) for h in hists]

    w1, w2, sc2 = _run_tcc(hists4, scal)
    (lovpart,) = _run_scd(w1.reshape(-1), w2.reshape(-1), lkey_f)

    out = _run_tce(sc2, lovpart)
    return out.reshape(())


# speedup vs baseline: 1.2989x; 1.2989x over previous
import jax
import jax.numpy as jnp
from jax import lax
from jax.experimental import pallas as pl
from jax.experimental.pallas import tpu as pltpu
from jax.experimental.pallas import tpu_sc as plsc

ALPHA = 0.3
BETA = 0.7
GAMMA = 1.33
SMOOTH = 1e-06
KEEP_RATIO = 0.3
LOVASZ_WEIGHT = 0.2

B = 16
TOTAL = 262144
K_ALL = max(1, int(TOTAL * KEEP_RATIO))
NBINS = 1024
BIN_OFF = (127 - 20) << 5
NEG_BIG = -3.0e38

CHUNKS_A = 8
ROWS_A = 64
LANES_A = 4096
NTEC = 32
PER_TEC = TOTAL // 2
SC_CHUNK = 2048
TBL_WORDS = 16 * NBINS


def _tca_body(x_ref, tgt_ref, tis_ref, okey_ref, lkey_ref, sc_ref, acc_ref):
    c = pl.program_id(1)

    x = x_ref[0]
    tgt = tgt_ref[0]
    tis = tis_ref[0]
    t = tgt.astype(jnp.float32)
    tisf = tis.astype(jnp.float32)

    ax = jnp.abs(x)
    bce = jnp.maximum(x, 0.0) - x * t + jnp.log(1.0 + jnp.exp(-ax))
    posm = (tgt == 1) & (tis == 1)
    negm = (tgt == 0) & (tis == 1)

    okey_ref[0] = jnp.where(negm, bce, 0.0)

    sig = 1.0 / (1.0 + jnp.exp(-x))
    e = 1.0 - x * (2.0 * t - 1.0)
    r = jnp.where(e > 0.0, e, 0.0)
    rb = lax.bitcast_convert_type(r, jnp.int32)
    rb = jnp.where(r > 0.0, (rb & ~1) | tgt, 0)
    lkey_ref[0] = lax.bitcast_convert_type(rb, jnp.float32)

    ri = lax.broadcasted_iota(jnp.int32, (ROWS_A // CHUNKS_A, LANES_A), 0)
    ci = lax.broadcasted_iota(jnp.int32, (ROWS_A // CHUNKS_A, LANES_A), 1)
    first = (ri == 0) & (ci == 0)

    pos_sum = jnp.sum(jnp.where(posm, bce, 0.0))
    n_pos = jnp.sum(jnp.where(posm, 1.0, 0.0))
    n_neg = jnp.sum(jnp.where(negm, 1.0, 0.0))
    tis_cnt = jnp.sum(tisf)
    tis_max = jnp.max(jnp.where(tis == 1, bce, NEG_BIG))
    bce0 = jnp.where(c == 0, jnp.sum(jnp.where(first, bce * tisf, 0.0)), NEG_BIG)
    tp = jnp.sum(sig * t)
    fn = jnp.sum((1.0 - sig) * t)
    fp = jnp.sum(sig * (1.0 - t))
    p_sum = jnp.sum(t)

    li = lax.broadcasted_iota(jnp.int32, (1, 128), 1)
    upd = jnp.zeros((1, 128), jnp.float32)
    for k, v in ((0, pos_sum), (1, n_pos), (2, n_neg), (3, tis_cnt),
                 (6, tp), (7, fn), (8, fp), (9, p_sum)):
        upd = upd + jnp.where(li == k, v, 0.0)
    mx = jnp.where(li == 4, tis_max, NEG_BIG) + jnp.where(li == 5, bce0 - NEG_BIG, 0.0)
    ismax = (li == 4) | (li == 5)

    @pl.when(c == 0)
    def _():
        acc_ref[...] = jnp.where(ismax, mx, upd)

    @pl.when(c > 0)
    def _():
        prev = acc_ref[...]
        acc_ref[...] = jnp.where(ismax, jnp.maximum(prev, mx), prev + upd)

    @pl.when(c == CHUNKS_A - 1)
    def _():
        sc_ref[0] = acc_ref[...]


def _run_tca(x, tgt, tis):
    bs_in = pl.BlockSpec((1, ROWS_A // CHUNKS_A, LANES_A), lambda s, c: (s, c, 0))
    bs_sc = pl.BlockSpec((1, 1, 128), lambda s, c: (s, 0, 0))
    return pl.pallas_call(
        _tca_body,
        grid=(B, CHUNKS_A),
        in_specs=[bs_in, bs_in, bs_in],
        out_specs=[bs_in, bs_in, bs_sc],
        out_shape=[
            jax.ShapeDtypeStruct((B, ROWS_A, LANES_A), jnp.float32),
            jax.ShapeDtypeStruct((B, ROWS_A, LANES_A), jnp.float32),
            jax.ShapeDtypeStruct((B, 1, 128), jnp.float32),
        ],
        scratch_shapes=[pltpu.VMEM((1, 128), jnp.float32)],
    )(x, tgt, tis)


def _bin16(v):
    bits = lax.bitcast_convert_type(v, jnp.int32)
    bn = jnp.minimum(jnp.maximum(bits >> 18, BIN_OFF), BIN_OFF + NBINS - 1) - BIN_OFF
    return bits, bn


NCH = PER_TEC // SC_CHUNK
UNROLL = 4


def _dbuf_stream(src, base, buf2, sem0, sem1, per_batch, init):
    def desc(slot, k, sem):
        return pltpu.make_async_copy(
            src.at[pl.ds(base + k * SC_CHUNK, SC_CHUNK)],
            buf2.at[pl.ds(slot * SC_CHUNK, SC_CHUNK)],
            sem,
        )

    desc(0, 0, sem0).start()

    def chunk(k, carry):
        @pl.when((k + 1 < NCH) & (k % 2 == 0))
        def _():
            desc(1, k + 1, sem1).start()

        @pl.when((k + 1 < NCH) & (k % 2 == 1))
        def _():
            desc(0, k + 1, sem0).start()

        @pl.when(k % 2 == 0)
        def _():
            desc(0, k, sem0).wait()

        @pl.when(k % 2 == 1)
        def _():
            desc(1, k, sem1).wait()

        off = (k % 2) * SC_CHUNK

        def inner(j, c2):
            vs = [buf2[pl.ds(off + (j * UNROLL + u) * 16, 16)] for u in range(UNROLL)]
            return per_batch(vs, c2)

        return lax.fori_loop(0, SC_CHUNK // (16 * UNROLL), inner, carry)

    return lax.fori_loop(0, NCH, chunk, init)


def _scb_body(okey, lkey, ocnt, osum, lpos, lneg,
              buf2, t0, t1, idxv, zbuf, s0m, s1m, s2m, s3m, sem0, sem1):
    core = lax.axis_index("c")
    sub = lax.axis_index("s")
    wid = sub * 2 + core
    base = wid * PER_TEC
    lane = lax.iota(jnp.int32, 16)
    ones = jnp.ones((16,), jnp.float32)
    zeros = jnp.zeros((16,), jnp.float32)

    def bidx(i, carry):
        idxv[pl.ds(i * 16, 16)] = jnp.zeros((16,), jnp.int32) + (sub * NBINS + i)
        return carry

    lax.fori_loop(0, TBL_WORDS // 16, bidx, 0)

    def zv(i, carry):
        zbuf[pl.ds(i * 16, 16)] = zeros
        return carry

    lax.fori_loop(0, NBINS // 16, zv, 0)
    row = pl.ds(sub * NBINS, NBINS)
    for spm in (s0m, s1m, s2m, s3m):
        pltpu.sync_copy(zbuf, spm.at[row])

    def zero_tables():
        def z(i, carry):
            for u in range(8):
                t0[pl.ds((i * 8 + u) * 16, 16)] = zeros
                t1[pl.ds((i * 8 + u) * 16, 16)] = zeros
            return carry

        lax.fori_loop(0, TBL_WORDS // 128, z, 0)

    def ohem_batch(vs, carry):
        idxs = [_bin16(v)[1] * 16 + lane for v in vs]
        for idx in idxs:
            plsc.addupdate_scatter(t0, [idx], ones)
        for idx, v in zip(idxs, vs):
            plsc.addupdate_scatter(t1, [idx], v)
        return carry

    def lov_batch(vs, carry):
        bits_l = [lax.bitcast_convert_type(v, jnp.int32) for v in vs]
        idxs = [_bin16(v)[1] * 16 + lane for v in vs]
        gfs = [(b & 1).astype(jnp.float32) for b in bits_l]
        nfs = [jnp.where((v > 0.0) & (gf == 0.0), 1.0, 0.0)
               for v, gf in zip(vs, gfs)]
        for idx, gf in zip(idxs, gfs):
            plsc.addupdate_scatter(t0, [idx], gf)
        for idx, nf in zip(idxs, nfs):
            plsc.addupdate_scatter(t1, [idx], nf)
        return carry

    zero_tables()
    _dbuf_stream(okey, base, buf2, sem0, sem1, ohem_batch, 0)
    pltpu.sync_copy(t0, s0m.at[idxv], add=True)
    pltpu.sync_copy(t1, s1m.at[idxv], add=True)

    zero_tables()
    _dbuf_stream(lkey, base, buf2, sem0, sem1, lov_batch, 0)
    pltpu.sync_copy(t0, s2m.at[idxv], add=True)
    pltpu.sync_copy(t1, s3m.at[idxv], add=True)

    out_off = pl.ds((core * 16 + sub) * NBINS, NBINS)
    pltpu.sync_copy(s0m.at[row], ocnt.at[out_off])
    pltpu.sync_copy(s1m.at[row], osum.at[out_off])
    pltpu.sync_copy(s2m.at[row], lpos.at[out_off])
    pltpu.sync_copy(s3m.at[row], lneg.at[out_off])


def _run_scb(okey_flat, lkey_flat):
    mesh = plsc.VectorSubcoreMesh(core_axis_name="c", subcore_axis_name="s")
    f = pl.kernel(
        _scb_body,
        mesh=mesh,
        compiler_params=pltpu.CompilerParams(needs_layout_passes=False),
        out_type=[jax.ShapeDtypeStruct((2 * B * NBINS,), jnp.float32)] * 4,
        scratch_types=[
            pltpu.VMEM((2 * SC_CHUNK,), jnp.float32),
            pltpu.VMEM((TBL_WORDS,), jnp.float32),
            pltpu.VMEM((TBL_WORDS,), jnp.float32),
            pltpu.VMEM((TBL_WORDS,), jnp.int32),
            pltpu.VMEM((NBINS,), jnp.float32),
            pltpu.VMEM_SHARED((B * NBINS,), jnp.float32),
            pltpu.VMEM_SHARED((B * NBINS,), jnp.float32),
            pltpu.VMEM_SHARED((B * NBINS,), jnp.float32),
            pltpu.VMEM_SHARED((B * NBINS,), jnp.float32),
            pltpu.SemaphoreType.DMA,
            pltpu.SemaphoreType.DMA,
        ],
    )
    return f(okey_flat, lkey_flat)


def _suffix_excl(X):
    ut = (lax.broadcasted_iota(jnp.int32, (128, 128), 0)
          > lax.broadcasted_iota(jnp.int32, (128, 128), 1)).astype(jnp.float32)
    within = jnp.dot(X, ut, preferred_element_type=jnp.float32)
    rows = jnp.sum(X, axis=1, keepdims=True)
    m8 = (lax.broadcasted_iota(jnp.int32, (8, 8), 1)
          > lax.broadcasted_iota(jnp.int32, (8, 8), 0)).astype(jnp.float32)
    above = jnp.dot(m8, rows, preferred_element_type=jnp.float32)
    return within + above


def _tcc_body(ocnt_ref, osum_ref, lpos_ref, lneg_ref, sc_ref, w1_ref, w2_ref, sc2_ref):
    red = lambda ref: jnp.sum(ref[...], axis=(0, 1))
    cnt = red(ocnt_ref)
    sm = red(osum_ref)
    lp = red(lpos_ref)
    ln = red(lneg_ref)

    srow = sc_ref[0]
    li = lax.broadcasted_iota(jnp.int32, (1, 128), 1)
    g = lambda k: jnp.sum(jnp.where(li == k, srow, 0.0))
    pos_sum, n_pos, num_neg = g(0), g(1), g(2)
    tis_cnt, tis_max, bce0 = g(3), g(4), g(5)
    tp, fn, fp, p = g(6), g(7), g(8), g(9)

    bi = lax.broadcasted_iota(jnp.int32, (8, 128), 0) * 128 + \
        lax.broadcasted_iota(jnp.int32, (8, 128), 1)
    is0 = (bi == 0).astype(jnp.float32)

    cnt = cnt - is0 * (jnp.float32(TOTAL) - num_neg)
    S = _suffix_excl(cnt)
    Ssum = _suffix_excl(sm)
    m = jnp.minimum(jnp.maximum(0.0, jnp.float32(K_ALL) - n_pos), num_neg)
    T = S + cnt
    mask = (S < m) & (m <= T)
    pick = lambda A: jnp.sum(jnp.where(mask, A, 0.0))
    S_t, cnt_t, sum_t, above_sum = pick(S), pick(cnt), pick(sm), pick(Ssum)
    lo_g = lax.bitcast_convert_type((bi + BIN_OFF) << 18, jnp.float32)
    hi_g = lax.bitcast_convert_type((bi + 1 + BIN_OFF) << 18, jnp.float32)
    lo, hi = pick(lo_g), pick(hi_g)
    mu = sum_t / jnp.maximum(cnt_t, 1e-30)
    h = jnp.maximum(0.0, jnp.minimum(hi - mu, mu - lo))
    kprime = m - S_t
    phi = kprime / jnp.maximum(cnt_t, 1e-30)
    neg_sum = above_sum + kprime * (mu + h * (1.0 - phi))
    kept = n_pos + m
    kept_loss = (pos_sum + neg_sum) / kept
    empty_loss = jnp.where(tis_cnt > 0, tis_max, bce0)
    ohem_i = jnp.where(kept == 0, empty_loss, kept_loss)

    tv = (tp + SMOOTH) / (tp + ALPHA * fn + BETA * fp + SMOOTH)
    omt = 1.0 - tv
    ft_i = jnp.where(omt > 0, jnp.exp(GAMMA * jnp.log(jnp.maximum(omt, 1e-38))), 0.0)

    CB = _suffix_excl(lp)
    NB = _suffix_excl(ln)
    n_neg_c = jnp.float32(TOTAL) - p
    a1 = p + NB + 0.5 * ln
    a2 = p + NB + 0.5 * (ln - 1.0)
    w1n = 1.0 / a1
    w2n = (p - CB - 0.5 * lp) / (a2 * (a2 + 1.0))
    w1a = (CB + 0.5 * (lp + 1.0)) / jnp.maximum(p, 1e-30)
    allpos = n_neg_c == 0
    w1_ref[0] = jnp.where(allpos, w1a, w1n)
    w2_ref[0] = jnp.where(allpos, 0.0, w2n)

    posb = jnp.where(p > 0, 1.0, 0.0)
    sc2_ref[0] = jnp.where(li == 0, ohem_i, 0.0) + \
        jnp.where(li == 1, ft_i, 0.0) + jnp.where(li == 2, posb, 0.0)


def _run_tcc(hists, scal):
    bs_h = pl.BlockSpec((2, 1, 8, 128), lambda s: (0, s, 0, 0))
    bs_s = pl.BlockSpec((1, 1, 128), lambda s: (s, 0, 0))
    bs_w = pl.BlockSpec((1, 8, 128), lambda s: (s, 0, 0))
    return pl.pallas_call(
        _tcc_body,
        grid=(B,),
        in_specs=[bs_h, bs_h, bs_h, bs_h, bs_s],
        out_specs=[bs_w, bs_w, bs_s],
        out_shape=[
            jax.ShapeDtypeStruct((B, 8, 128), jnp.float32),
            jax.ShapeDtypeStruct((B, 8, 128), jnp.float32),
            jax.ShapeDtypeStruct((B, 1, 128), jnp.float32),
        ],
    )(*hists, scal)


def _scd_body(w1f, w2f, lkey, lout, w1v, w2v, w1r, w2r, buf2, accv, sem0, sem1):
    wid = lax.axis_index("s") * 2 + lax.axis_index("c")
    s = wid // 2
    base = wid * PER_TEC
    lane = lax.iota(jnp.int32, 16)
    pltpu.sync_copy(w1f.at[pl.ds(s * NBINS, NBINS)], w1v)
    pltpu.sync_copy(w2f.at[pl.ds(s * NBINS, NBINS)], w2v)

    def rep(i, carry):
        a = w1v[pl.ds(i * 16, 16)]
        b = w2v[pl.ds(i * 16, 16)]
        bins16 = (jnp.zeros((16,), jnp.int32) + i * 16 + lane) * 16
        for l in range(16):
            plsc.store_scatter(w1r, [bins16 + l], a)
            plsc.store_scatter(w2r, [bins16 + l], b)
        return carry

    lax.fori_loop(0, NBINS // 16, rep, 0)

    def lov_batch(vs, acc):
        bits_l = [lax.bitcast_convert_type(v, jnp.int32) for v in vs]
        idxs = [_bin16(v)[1] * 16 + lane for v in vs]
        was = [plsc.load_gather(w1r, [idx]) for idx in idxs]
        wbs = [plsc.load_gather(w2r, [idx]) for idx in idxs]
        for v, b, wa, wb in zip(vs, bits_l, was, wbs):
            acc = acc + v * jnp.where((b & 1) == 1, wa, wb)
        return acc

    acc = _dbuf_stream(lkey, base, buf2, sem0, sem1, lov_batch,
                       jnp.zeros((16,), jnp.float32))
    accv[pl.ds(0, 16)] = acc
    pltpu.sync_copy(accv, lout.at[pl.ds(wid * 16, 16)])


def _run_scd(w1, w2, lkey_flat):
    mesh = plsc.VectorSubcoreMesh(core_axis_name="c", subcore_axis_name="s")
    f = pl.kernel(
        _scd_body,
        mesh=mesh,
        compiler_params=pltpu.CompilerParams(needs_layout_passes=False),
        out_type=[jax.ShapeDtypeStruct((NTEC * 16,), jnp.float32)],
        scratch_types=[
            pltpu.VMEM((NBINS,), jnp.float32),
            pltpu.VMEM((NBINS,), jnp.float32),
            pltpu.VMEM((TBL_WORDS,), jnp.float32),
            pltpu.VMEM((TBL_WORDS,), jnp.float32),
            pltpu.VMEM((2 * SC_CHUNK,), jnp.float32),
            pltpu.VMEM((16,), jnp.float32),
            pltpu.SemaphoreType.DMA,
            pltpu.SemaphoreType.DMA,
        ],
    )
    return f(w1, w2, lkey_flat)


def _tce_body(sc2_ref, lov_ref, out_ref):
    sc2 = sc2_ref[...]
    li = lax.broadcasted_iota(jnp.int32, (B, 1, 128), 2)
    col = lambda k: jnp.sum(jnp.where(li == k, sc2, 0.0), axis=(1, 2))
    ohem_i, ft_i, posb = col(0), col(1), col(2)
    lov_i = jnp.sum(lov_ref[...], axis=1)
    n_pos_b = jnp.sum(posb)
    ohem_term = jnp.sum(ohem_i) / jnp.float32(B)
    ft_term = jnp.sum(jnp.where(posb > 0, ft_i, 0.0)) / n_pos_b
    lov_term = jnp.sum(jnp.where(posb > 0, lov_i, 0.0)) / n_pos_b
    full = ohem_term + ft_term + LOVASZ_WEIGHT * lov_term
    out_ref[...] = jnp.broadcast_to(jnp.where(n_pos_b > 0, full, ohem_term), (1, 1))


def _run_tce(sc2, lovpart):
    return pl.pallas_call(
        _tce_body,
        out_shape=jax.ShapeDtypeStruct((1, 1), jnp.float32),
    )(sc2, lovpart.reshape(B, NTEC * 16 // B))


def kernel(logits, targets, tissue_mask):
    x = logits.reshape(B, ROWS_A, LANES_A)
    tgt = targets.reshape(B, ROWS_A, LANES_A)
    tis = tissue_mask.reshape(B, ROWS_A, LANES_A)

    okey, lkey, scal = _run_tca(x, tgt, tis)
    okey_f = okey.reshape(-1)
    lkey_f = lkey.reshape(-1)

    hists = _run_scb(okey_f, lkey_f)
    hists4 = [h.reshape(2, B, 8, 128) for h in hists]

    w1, w2, sc2 = _run_tcc(hists4, scal)
    (lovpart,) = _run_scd(w1.reshape(-1), w2.reshape(-1), lkey_f)

    out = _run_tce(sc2, lovpart)
    return out.reshape(())


# speedup vs baseline: 1.3273x; 1.0219x over previous
import jax
import jax.numpy as jnp
from jax import lax
from jax.experimental import pallas as pl
from jax.experimental.pallas import tpu as pltpu
from jax.experimental.pallas import tpu_sc as plsc

ALPHA = 0.3
BETA = 0.7
GAMMA = 1.33
SMOOTH = 1e-06
KEEP_RATIO = 0.3
LOVASZ_WEIGHT = 0.2

B = 16
TOTAL = 262144
K_ALL = max(1, int(TOTAL * KEEP_RATIO))
NBINS = 1024
BIN_OFF = (127 - 20) << 5
NEG_BIG = -3.0e38

CHUNKS_A = 8
ROWS_A = 64
LANES_A = 4096
NTEC = 32
PER_TEC = TOTAL // 2
SC_CHUNK = 2048
TBL_WORDS = 16 * NBINS


def _tca_body(x_ref, tgt_ref, tis_ref, okey_ref, lkey_ref, sc_ref, acc_ref):
    c = pl.program_id(1)

    x = x_ref[0]
    tgt = tgt_ref[0]
    tis = tis_ref[0]
    t = tgt.astype(jnp.float32)
    tisf = tis.astype(jnp.float32)

    ax = jnp.abs(x)
    enax = jnp.exp(-ax)
    bce = jnp.maximum(x, 0.0) - x * t + jnp.log(1.0 + enax)
    posm = (tgt == 1) & (tis == 1)
    negm = (tgt == 0) & (tis == 1)

    okey_ref[0] = jnp.where(negm, bce, 0.0)

    sig = jnp.where(x >= 0, 1.0, enax) / (1.0 + enax)
    e = 1.0 - x * (2.0 * t - 1.0)
    r = jnp.where(e > 0.0, e, 0.0)
    rb = lax.bitcast_convert_type(r, jnp.int32)
    rb = jnp.where(r > 0.0, (rb & ~1) | tgt, 0)
    lkey_ref[0] = lax.bitcast_convert_type(rb, jnp.float32)

    ri = lax.broadcasted_iota(jnp.int32, (ROWS_A // CHUNKS_A, LANES_A), 0)
    ci = lax.broadcasted_iota(jnp.int32, (ROWS_A // CHUNKS_A, LANES_A), 1)
    first = (ri == 0) & (ci == 0)

    pos_sum = jnp.sum(jnp.where(posm, bce, 0.0))
    n_pos = jnp.sum(jnp.where(posm, 1.0, 0.0))
    n_neg = jnp.sum(jnp.where(negm, 1.0, 0.0))
    tis_cnt = jnp.sum(tisf)
    tis_max = jnp.max(jnp.where(tis == 1, bce, NEG_BIG))
    bce0 = jnp.where(c == 0, jnp.sum(jnp.where(first, bce * tisf, 0.0)), NEG_BIG)
    tp = jnp.sum(sig * t)
    fn = jnp.sum((1.0 - sig) * t)
    fp = jnp.sum(sig * (1.0 - t))
    p_sum = jnp.sum(t)

    li = lax.broadcasted_iota(jnp.int32, (1, 128), 1)
    upd = jnp.zeros((1, 128), jnp.float32)
    for k, v in ((0, pos_sum), (1, n_pos), (2, n_neg), (3, tis_cnt),
                 (6, tp), (7, fn), (8, fp), (9, p_sum)):
        upd = upd + jnp.where(li == k, v, 0.0)
    mx = jnp.where(li == 4, tis_max, NEG_BIG) + jnp.where(li == 5, bce0 - NEG_BIG, 0.0)
    ismax = (li == 4) | (li == 5)

    @pl.when(c == 0)
    def _():
        acc_ref[...] = jnp.where(ismax, mx, upd)

    @pl.when(c > 0)
    def _():
        prev = acc_ref[...]
        acc_ref[...] = jnp.where(ismax, jnp.maximum(prev, mx), prev + upd)

    @pl.when(c == CHUNKS_A - 1)
    def _():
        sc_ref[0] = acc_ref[...]


def _run_tca(x, tgt, tis):
    bs_in = pl.BlockSpec((1, ROWS_A // CHUNKS_A, LANES_A), lambda s, c: (s, c, 0))
    bs_sc = pl.BlockSpec((1, 1, 128), lambda s, c: (s, 0, 0))
    return pl.pallas_call(
        _tca_body,
        grid=(B, CHUNKS_A),
        in_specs=[bs_in, bs_in, bs_in],
        out_specs=[bs_in, bs_in, bs_sc],
        out_shape=[
            jax.ShapeDtypeStruct((B, ROWS_A, LANES_A), jnp.float32),
            jax.ShapeDtypeStruct((B, ROWS_A, LANES_A), jnp.float32),
            jax.ShapeDtypeStruct((B, 1, 128), jnp.float32),
        ],
        scratch_shapes=[pltpu.VMEM((1, 128), jnp.float32)],
    )(x, tgt, tis)


def _bin16(v):
    bits = lax.bitcast_convert_type(v, jnp.int32)
    bn = jnp.minimum(jnp.maximum(bits >> 18, BIN_OFF), BIN_OFF + NBINS - 1) - BIN_OFF
    return bits, bn


NCH = PER_TEC // SC_CHUNK
UNROLL = 8


def _dbuf_stream(src, base, buf2, sem0, sem1, per_batch, init):
    def desc(slot, k, sem):
        return pltpu.make_async_copy(
            src.at[pl.ds(base + k * SC_CHUNK, SC_CHUNK)],
            buf2.at[pl.ds(slot * SC_CHUNK, SC_CHUNK)],
            sem,
        )

    desc(0, 0, sem0).start()

    def chunk(k, carry):
        @pl.when((k + 1 < NCH) & (k % 2 == 0))
        def _():
            desc(1, k + 1, sem1).start()

        @pl.when((k + 1 < NCH) & (k % 2 == 1))
        def _():
            desc(0, k + 1, sem0).start()

        @pl.when(k % 2 == 0)
        def _():
            desc(0, k, sem0).wait()

        @pl.when(k % 2 == 1)
        def _():
            desc(1, k, sem1).wait()

        off = (k % 2) * SC_CHUNK

        def inner(j, c2):
            vs = [buf2[pl.ds(off + (j * UNROLL + u) * 16, 16)] for u in range(UNROLL)]
            return per_batch(vs, c2)

        return lax.fori_loop(0, SC_CHUNK // (16 * UNROLL), inner, carry)

    return lax.fori_loop(0, NCH, chunk, init)


def _scb_body(okey, lkey, ocnt, osum, lpos, lneg,
              buf2, t0, t1, idxv, zbuf, s0m, s1m, s2m, s3m, sem0, sem1):
    core = lax.axis_index("c")
    sub = lax.axis_index("s")
    wid = sub * 2 + core
    base = wid * PER_TEC
    lane = lax.iota(jnp.int32, 16)
    ones = jnp.ones((16,), jnp.float32)
    zeros = jnp.zeros((16,), jnp.float32)

    def bidx(i, carry):
        idxv[pl.ds(i * 16, 16)] = jnp.zeros((16,), jnp.int32) + (sub * NBINS + i)
        return carry

    lax.fori_loop(0, TBL_WORDS // 16, bidx, 0)

    def zv(i, carry):
        zbuf[pl.ds(i * 16, 16)] = zeros
        return carry

    lax.fori_loop(0, NBINS // 16, zv, 0)
    row = pl.ds(sub * NBINS, NBINS)
    for spm in (s0m, s1m, s2m, s3m):
        pltpu.sync_copy(zbuf, spm.at[row])

    def zero_tables():
        def z(i, carry):
            for u in range(8):
                t0[pl.ds((i * 8 + u) * 16, 16)] = zeros
                t1[pl.ds((i * 8 + u) * 16, 16)] = zeros
            return carry

        lax.fori_loop(0, TBL_WORDS // 128, z, 0)

    def ohem_batch(vs, carry):
        idxs = [_bin16(v)[1] * 16 + lane for v in vs]
        for idx in idxs:
            plsc.addupdate_scatter(t0, [idx], ones)
        for idx, v in zip(idxs, vs):
            plsc.addupdate_scatter(t1, [idx], v)
        return carry

    def lov_batch(vs, carry):
        bits_l = [lax.bitcast_convert_type(v, jnp.int32) for v in vs]
        idxs = [_bin16(v)[1] * 16 + lane for v in vs]
        gfs = [(b & 1).astype(jnp.float32) for b in bits_l]
        nfs = [jnp.where((v > 0.0) & (gf == 0.0), 1.0, 0.0)
               for v, gf in zip(vs, gfs)]
        for idx, gf in zip(idxs, gfs):
            plsc.addupdate_scatter(t0, [idx], gf)
        for idx, nf in zip(idxs, nfs):
            plsc.addupdate_scatter(t1, [idx], nf)
        return carry

    zero_tables()
    _dbuf_stream(okey, base, buf2, sem0, sem1, ohem_batch, 0)
    pltpu.sync_copy(t0, s0m.at[idxv], add=True)
    pltpu.sync_copy(t1, s1m.at[idxv], add=True)

    zero_tables()
    _dbuf_stream(lkey, base, buf2, sem0, sem1, lov_batch, 0)
    pltpu.sync_copy(t0, s2m.at[idxv], add=True)
    pltpu.sync_copy(t1, s3m.at[idxv], add=True)

    out_off = pl.ds((core * 16 + sub) * NBINS, NBINS)
    pltpu.sync_copy(s0m.at[row], ocnt.at[out_off])
    pltpu.sync_copy(s1m.at[row], osum.at[out_off])
    pltpu.sync_copy(s2m.at[row], lpos.at[out_off])
    pltpu.sync_copy(s3m.at[row], lneg.at[out_off])


def _run_scb(okey_flat, lkey_flat):
    mesh = plsc.VectorSubcoreMesh(core_axis_name="c", subcore_axis_name="s")
    f = pl.kernel(
        _scb_body,
        mesh=mesh,
        compiler_params=pltpu.CompilerParams(needs_layout_passes=False),
        out_type=[jax.ShapeDtypeStruct((2 * B * NBINS,), jnp.float32)] * 4,
        scratch_types=[
            pltpu.VMEM((2 * SC_CHUNK,), jnp.float32),
            pltpu.VMEM((TBL_WORDS,), jnp.float32),
            pltpu.VMEM((TBL_WORDS,), jnp.float32),
            pltpu.VMEM((TBL_WORDS,), jnp.int32),
            pltpu.VMEM((NBINS,), jnp.float32),
            pltpu.VMEM_SHARED((B * NBINS,), jnp.float32),
            pltpu.VMEM_SHARED((B * NBINS,), jnp.float32),
            pltpu.VMEM_SHARED((B * NBINS,), jnp.float32),
            pltpu.VMEM_SHARED((B * NBINS,), jnp.float32),
            pltpu.SemaphoreType.DMA,
            pltpu.SemaphoreType.DMA,
        ],
    )
    return f(okey_flat, lkey_flat)


def _suffix_excl(X):
    ut = (lax.broadcasted_iota(jnp.int32, (128, 128), 0)
          > lax.broadcasted_iota(jnp.int32, (128, 128), 1)).astype(jnp.float32)
    within = jnp.dot(X, ut, preferred_element_type=jnp.float32)
    rows = jnp.sum(X, axis=1, keepdims=True)
    m8 = (lax.broadcasted_iota(jnp.int32, (8, 8), 1)
          > lax.broadcasted_iota(jnp.int32, (8, 8), 0)).astype(jnp.float32)
    above = jnp.dot(m8, rows, preferred_element_type=jnp.float32)
    return within + above


def _tcc_body(ocnt_ref, osum_ref, lpos_ref, lneg_ref, sc_ref, w1_ref, w2_ref, sc2_ref):
    red = lambda ref: jnp.sum(ref[...], axis=(0, 1))
    cnt = red(ocnt_ref)
    sm = red(osum_ref)
    lp = red(lpos_ref)
    ln = red(lneg_ref)

    srow = sc_ref[0]
    li = lax.broadcasted_iota(jnp.int32, (1, 128), 1)
    g = lambda k: jnp.sum(jnp.where(li == k, srow, 0.0))
    pos_sum, n_pos, num_neg = g(0), g(1), g(2)
    tis_cnt, tis_max, bce0 = g(3), g(4), g(5)
    tp, fn, fp, p = g(6), g(7), g(8), g(9)

    bi = lax.broadcasted_iota(jnp.int32, (8, 128), 0) * 128 + \
        lax.broadcasted_iota(jnp.int32, (8, 128), 1)
    is0 = (bi == 0).astype(jnp.float32)

    cnt = cnt - is0 * (jnp.float32(TOTAL) - num_neg)
    S = _suffix_excl(cnt)
    Ssum = _suffix_excl(sm)
    m = jnp.minimum(jnp.maximum(0.0, jnp.float32(K_ALL) - n_pos), num_neg)
    T = S + cnt
    mask = (S < m) & (m <= T)
    pick = lambda A: jnp.sum(jnp.where(mask, A, 0.0))
    S_t, cnt_t, sum_t, above_sum = pick(S), pick(cnt), pick(sm), pick(Ssum)
    lo_g = lax.bitcast_convert_type((bi + BIN_OFF) << 18, jnp.float32)
    hi_g = lax.bitcast_convert_type((bi + 1 + BIN_OFF) << 18, jnp.float32)
    lo, hi = pick(lo_g), pick(hi_g)
    mu = sum_t / jnp.maximum(cnt_t, 1e-30)
    h = jnp.maximum(0.0, jnp.minimum(hi - mu, mu - lo))
    kprime = m - S_t
    phi = kprime / jnp.maximum(cnt_t, 1e-30)
    neg_sum = above_sum + kprime * (mu + h * (1.0 - phi))
    kept = n_pos + m
    kept_loss = (pos_sum + neg_sum) / kept
    empty_loss = jnp.where(tis_cnt > 0, tis_max, bce0)
    ohem_i = jnp.where(kept == 0, empty_loss, kept_loss)

    tv = (tp + SMOOTH) / (tp + ALPHA * fn + BETA * fp + SMOOTH)
    omt = 1.0 - tv
    ft_i = jnp.where(omt > 0, jnp.exp(GAMMA * jnp.log(jnp.maximum(omt, 1e-38))), 0.0)

    CB = _suffix_excl(lp)
    NB = _suffix_excl(ln)
    n_neg_c = jnp.float32(TOTAL) - p
    a1 = p + NB + 0.5 * ln
    a2 = p + NB + 0.5 * (ln - 1.0)
    w1n = 1.0 / a1
    w2n = (p - CB - 0.5 * lp) / (a2 * (a2 + 1.0))
    w1a = (CB + 0.5 * (lp + 1.0)) / jnp.maximum(p, 1e-30)
    allpos = n_neg_c == 0
    w1_ref[0] = jnp.where(allpos, w1a, w1n)
    w2_ref[0] = jnp.where(allpos, 0.0, w2n)

    posb = jnp.where(p > 0, 1.0, 0.0)
    sc2_ref[0] = jnp.where(li == 0, ohem_i, 0.0) + \
        jnp.where(li == 1, ft_i, 0.0) + jnp.where(li == 2, posb, 0.0)


def _run_tcc(hists, scal):
    bs_h = pl.BlockSpec((2, 1, 8, 128), lambda s: (0, s, 0, 0))
    bs_s = pl.BlockSpec((1, 1, 128), lambda s: (s, 0, 0))
    bs_w = pl.BlockSpec((1, 8, 128), lambda s: (s, 0, 0))
    return pl.pallas_call(
        _tcc_body,
        grid=(B,),
        in_specs=[bs_h, bs_h, bs_h, bs_h, bs_s],
        out_specs=[bs_w, bs_w, bs_s],
        out_shape=[
            jax.ShapeDtypeStruct((B, 8, 128), jnp.float32),
            jax.ShapeDtypeStruct((B, 8, 128), jnp.float32),
            jax.ShapeDtypeStruct((B, 1, 128), jnp.float32),
        ],
    )(*hists, scal)


def _scd_body(w1f, w2f, lkey, lout, w1v, w2v, w1r, w2r, buf2, accv, sem0, sem1):
    wid = lax.axis_index("s") * 2 + lax.axis_index("c")
    s = wid // 2
    base = wid * PER_TEC
    lane = lax.iota(jnp.int32, 16)
    pltpu.sync_copy(w1f.at[pl.ds(s * NBINS, NBINS)], w1v)
    pltpu.sync_copy(w2f.at[pl.ds(s * NBINS, NBINS)], w2v)

    def rep(i, carry):
        a = w1v[pl.ds(i * 16, 16)]
        b = w2v[pl.ds(i * 16, 16)]
        bins16 = (jnp.zeros((16,), jnp.int32) + i * 16 + lane) * 16
        for l in range(16):
            plsc.store_scatter(w1r, [bins16 + l], a)
            plsc.store_scatter(w2r, [bins16 + l], b)
        return carry

    lax.fori_loop(0, NBINS // 16, rep, 0)

    def lov_batch(vs, acc):
        bits_l = [lax.bitcast_convert_type(v, jnp.int32) for v in vs]
        idxs = [_bin16(v)[1] * 16 + lane for v in vs]
        was = [plsc.load_gather(w1r, [idx]) for idx in idxs]
        wbs = [plsc.load_gather(w2r, [idx]) for idx in idxs]
        for v, b, wa, wb in zip(vs, bits_l, was, wbs):
            acc = acc + v * jnp.where((b & 1) == 1, wa, wb)
        return acc

    acc = _dbuf_stream(lkey, base, buf2, sem0, sem1, lov_batch,
                       jnp.zeros((16,), jnp.float32))
    accv[pl.ds(0, 16)] = acc
    pltpu.sync_copy(accv, lout.at[pl.ds(wid * 16, 16)])


def _run_scd(w1, w2, lkey_flat):
    mesh = plsc.VectorSubcoreMesh(core_axis_name="c", subcore_axis_name="s")
    f = pl.kernel(
        _scd_body,
        mesh=mesh,
        compiler_params=pltpu.CompilerParams(needs_layout_passes=False),
        out_type=[jax.ShapeDtypeStruct((NTEC * 16,), jnp.float32)],
        scratch_types=[
            pltpu.VMEM((NBINS,), jnp.float32),
            pltpu.VMEM((NBINS,), jnp.float32),
            pltpu.VMEM((TBL_WORDS,), jnp.float32),
            pltpu.VMEM((TBL_WORDS,), jnp.float32),
            pltpu.VMEM((2 * SC_CHUNK,), jnp.float32),
            pltpu.VMEM((16,), jnp.float32),
            pltpu.SemaphoreType.DMA,
            pltpu.SemaphoreType.DMA,
        ],
    )
    return f(w1, w2, lkey_flat)


def _tce_body(sc2_ref, lov_ref, out_ref):
    sc2 = sc2_ref[...]
    li = lax.broadcasted_iota(jnp.int32, (B, 1, 128), 2)
    col = lambda k: jnp.sum(jnp.where(li == k, sc2, 0.0), axis=(1, 2))
    ohem_i, ft_i, posb = col(0), col(1), col(2)
    lov_i = jnp.sum(lov_ref[...], axis=1)
    n_pos_b = jnp.sum(posb)
    ohem_term = jnp.sum(ohem_i) / jnp.float32(B)
    ft_term = jnp.sum(jnp.where(posb > 0, ft_i, 0.0)) / n_pos_b
    lov_term = jnp.sum(jnp.where(posb > 0, lov_i, 0.0)) / n_pos_b
    full = ohem_term + ft_term + LOVASZ_WEIGHT * lov_term
    out_ref[...] = jnp.broadcast_to(jnp.where(n_pos_b > 0, full, ohem_term), (1, 1))


def _run_tce(sc2, lovpart):
    return pl.pallas_call(
        _tce_body,
        out_shape=jax.ShapeDtypeStruct((1, 1), jnp.float32),
    )(sc2, lovpart.reshape(B, NTEC * 16 // B))


def kernel(logits, targets, tissue_mask):
    x = logits.reshape(B, ROWS_A, LANES_A)
    tgt = targets.reshape(B, ROWS_A, LANES_A)
    tis = tissue_mask.reshape(B, ROWS_A, LANES_A)

    okey, lkey, scal = _run_tca(x, tgt, tis)
    okey_f = okey.reshape(-1)
    lkey_f = lkey.reshape(-1)

    hists = _run_scb(okey_f, lkey_f)
    hists4 = [h.reshape(2, B, 8, 128) for h in hists]

    w1, w2, sc2 = _run_tcc(hists4, scal)
    (lovpart,) = _run_scd(w1.reshape(-1), w2.reshape(-1), lkey_f)

    out = _run_tce(sc2, lovpart)
    return out.reshape(())


# speedup vs baseline: 1.4537x; 1.0952x over previous
import jax
import jax.numpy as jnp
from jax import lax
from jax.experimental import pallas as pl
from jax.experimental.pallas import tpu as pltpu
from jax.experimental.pallas import tpu_sc as plsc

ALPHA = 0.3
BETA = 0.7
GAMMA = 1.33
SMOOTH = 1e-06
KEEP_RATIO = 0.3
LOVASZ_WEIGHT = 0.2

B = 16
TOTAL = 262144
K_ALL = max(1, int(TOTAL * KEEP_RATIO))
NBINS = 1024
BIN_OFF = (127 - 20) << 5
NEG_BIG = -3.0e38

CHUNKS_A = 8
ROWS_A = 64
LANES_A = 4096
NTEC = 32
PER_TEC = TOTAL // 2
SC_CHUNK = 2048
TBL_WORDS = 16 * NBINS


def _tca_body(x_ref, tgt_ref, tis_ref, okey_ref, lkey_ref, sc_ref, acc_ref):
    c = pl.program_id(1)

    x = x_ref[0]
    tgt = tgt_ref[0]
    tis = tis_ref[0]
    t = tgt.astype(jnp.float32)
    tisf = tis.astype(jnp.float32)

    ax = jnp.abs(x)
    enax = jnp.exp(-ax)
    bce = jnp.maximum(x, 0.0) - x * t + jnp.log(1.0 + enax)
    posm = (tgt == 1) & (tis == 1)
    negm = (tgt == 0) & (tis == 1)

    okey_ref[0] = jnp.where(negm, bce, 0.0)

    sig = jnp.where(x >= 0, 1.0, enax) / (1.0 + enax)
    e = 1.0 - x * (2.0 * t - 1.0)
    r = jnp.where(e > 0.0, e, 0.0)
    rb = lax.bitcast_convert_type(r, jnp.int32)
    rb = jnp.where(r > 0.0, (rb & ~1) | tgt, 0)
    lkey_ref[0] = lax.bitcast_convert_type(rb, jnp.float32)

    ri = lax.broadcasted_iota(jnp.int32, (ROWS_A // CHUNKS_A, LANES_A), 0)
    ci = lax.broadcasted_iota(jnp.int32, (ROWS_A // CHUNKS_A, LANES_A), 1)
    first = (ri == 0) & (ci == 0)

    pos_sum = jnp.sum(jnp.where(posm, bce, 0.0))
    n_pos = jnp.sum(jnp.where(posm, 1.0, 0.0))
    n_neg = jnp.sum(jnp.where(negm, 1.0, 0.0))
    tis_cnt = jnp.sum(tisf)
    tis_max = jnp.max(jnp.where(tis == 1, bce, NEG_BIG))
    bce0 = jnp.where(c == 0, jnp.sum(jnp.where(first, bce * tisf, 0.0)), NEG_BIG)
    tp = jnp.sum(sig * t)
    fn = jnp.sum((1.0 - sig) * t)
    fp = jnp.sum(sig * (1.0 - t))
    p_sum = jnp.sum(t)

    li = lax.broadcasted_iota(jnp.int32, (1, 128), 1)
    upd = jnp.zeros((1, 128), jnp.float32)
    for k, v in ((0, pos_sum), (1, n_pos), (2, n_neg), (3, tis_cnt),
                 (6, tp), (7, fn), (8, fp), (9, p_sum)):
        upd = upd + jnp.where(li == k, v, 0.0)
    mx = jnp.where(li == 4, tis_max, NEG_BIG) + jnp.where(li == 5, bce0 - NEG_BIG, 0.0)
    ismax = (li == 4) | (li == 5)

    @pl.when(c == 0)
    def _():
        acc_ref[...] = jnp.where(ismax, mx, upd)

    @pl.when(c > 0)
    def _():
        prev = acc_ref[...]
        acc_ref[...] = jnp.where(ismax, jnp.maximum(prev, mx), prev + upd)

    @pl.when(c == CHUNKS_A - 1)
    def _():
        sc_ref[0] = acc_ref[...]


def _run_tca(x, tgt, tis):
    bs_in = pl.BlockSpec((1, ROWS_A // CHUNKS_A, LANES_A), lambda s, c: (s, c, 0))
    bs_sc = pl.BlockSpec((1, 1, 128), lambda s, c: (s, 0, 0))
    return pl.pallas_call(
        _tca_body,
        grid=(B, CHUNKS_A),
        in_specs=[bs_in, bs_in, bs_in],
        out_specs=[bs_in, bs_in, bs_sc],
        out_shape=[
            jax.ShapeDtypeStruct((B, ROWS_A, LANES_A), jnp.float32),
            jax.ShapeDtypeStruct((B, ROWS_A, LANES_A), jnp.float32),
            jax.ShapeDtypeStruct((B, 1, 128), jnp.float32),
        ],
        scratch_shapes=[pltpu.VMEM((1, 128), jnp.float32)],
    )(x, tgt, tis)


def _bin16(v):
    bits = lax.bitcast_convert_type(v, jnp.int32)
    bn = jnp.minimum(jnp.maximum(bits >> 18, BIN_OFF), BIN_OFF + NBINS - 1) - BIN_OFF
    return bits, bn


NCH = PER_TEC // SC_CHUNK
UNROLL = 8


def _dbuf_stream(slicer, buf2, sem0, sem1, per_batch, init):
    def desc(slot, k, sem):
        return pltpu.make_async_copy(
            slicer(k),
            buf2.at[pl.ds(slot * SC_CHUNK, SC_CHUNK)],
            sem,
        )

    desc(0, 0, sem0).start()

    def chunk(k, carry):
        @pl.when((k + 1 < NCH) & (k % 2 == 0))
        def _():
            desc(1, k + 1, sem1).start()

        @pl.when((k + 1 < NCH) & (k % 2 == 1))
        def _():
            desc(0, k + 1, sem0).start()

        @pl.when(k % 2 == 0)
        def _():
            desc(0, k, sem0).wait()

        @pl.when(k % 2 == 1)
        def _():
            desc(1, k, sem1).wait()

        off = (k % 2) * SC_CHUNK

        def inner(j, c2):
            vs = [buf2[pl.ds(off + (j * UNROLL + u) * 16, 16)] for u in range(UNROLL)]
            return per_batch(vs, c2)

        return lax.fori_loop(0, SC_CHUNK // (16 * UNROLL), inner, carry)

    return lax.fori_loop(0, NCH, chunk, init)


def _scb_body(okey, lkey, ocnt, osum, lpos, lneg,
              buf2, t0, t1, idxv, zbuf, s0m, s1m, s2m, s3m, sem0, sem1):
    core = lax.axis_index("c")
    sub = lax.axis_index("s")
    lane = lax.iota(jnp.int32, 16)

    def slicer(src):
        def sl(k):
            return src.at[sub, core * 32 + k // 2, pl.ds((k % 2) * SC_CHUNK, SC_CHUNK)]

        return sl
    ones = jnp.ones((16,), jnp.float32)
    zeros = jnp.zeros((16,), jnp.float32)

    def bidx(i, carry):
        idxv[pl.ds(i * 16, 16)] = jnp.zeros((16,), jnp.int32) + (sub * NBINS + i)
        return carry

    lax.fori_loop(0, TBL_WORDS // 16, bidx, 0)

    def zv(i, carry):
        zbuf[pl.ds(i * 16, 16)] = zeros
        return carry

    lax.fori_loop(0, NBINS // 16, zv, 0)
    row = pl.ds(sub * NBINS, NBINS)
    for spm in (s0m, s1m, s2m, s3m):
        pltpu.sync_copy(zbuf, spm.at[row])

    def zero_tables():
        def z(i, carry):
            for u in range(8):
                t0[pl.ds((i * 8 + u) * 16, 16)] = zeros
                t1[pl.ds((i * 8 + u) * 16, 16)] = zeros
            return carry

        lax.fori_loop(0, TBL_WORDS // 128, z, 0)

    def ohem_batch(vs, carry):
        idxs = [_bin16(v)[1] * 16 + lane for v in vs]
        for idx in idxs:
            plsc.addupdate_scatter(t0, [idx], ones)
        for idx, v in zip(idxs, vs):
            plsc.addupdate_scatter(t1, [idx], v)
        return carry

    def lov_batch(vs, carry):
        bits_l = [lax.bitcast_convert_type(v, jnp.int32) for v in vs]
        idxs = [_bin16(v)[1] * 16 + lane for v in vs]
        gfs = [(b & 1).astype(jnp.float32) for b in bits_l]
        nfs = [jnp.where((v > 0.0) & (gf == 0.0), 1.0, 0.0)
               for v, gf in zip(vs, gfs)]
        for idx, gf in zip(idxs, gfs):
            plsc.addupdate_scatter(t0, [idx], gf)
        for idx, nf in zip(idxs, nfs):
            plsc.addupdate_scatter(t1, [idx], nf)
        return carry

    zero_tables()
    _dbuf_stream(slicer(okey), buf2, sem0, sem1, ohem_batch, 0)
    pltpu.sync_copy(t0, s0m.at[idxv], add=True)
    pltpu.sync_copy(t1, s1m.at[idxv], add=True)

    zero_tables()
    _dbuf_stream(slicer(lkey), buf2, sem0, sem1, lov_batch, 0)
    pltpu.sync_copy(t0, s2m.at[idxv], add=True)
    pltpu.sync_copy(t1, s3m.at[idxv], add=True)

    out_off = pl.ds((core * 16 + sub) * NBINS, NBINS)
    pltpu.sync_copy(s0m.at[row], ocnt.at[out_off])
    pltpu.sync_copy(s1m.at[row], osum.at[out_off])
    pltpu.sync_copy(s2m.at[row], lpos.at[out_off])
    pltpu.sync_copy(s3m.at[row], lneg.at[out_off])


def _run_scb(okey_flat, lkey_flat):
    mesh = plsc.VectorSubcoreMesh(core_axis_name="c", subcore_axis_name="s")
    f = pl.kernel(
        _scb_body,
        mesh=mesh,
        compiler_params=pltpu.CompilerParams(needs_layout_passes=False),
        out_type=[jax.ShapeDtypeStruct((2 * B * NBINS,), jnp.float32)] * 4,
        scratch_types=[
            pltpu.VMEM((2 * SC_CHUNK,), jnp.float32),
            pltpu.VMEM((TBL_WORDS,), jnp.float32),
            pltpu.VMEM((TBL_WORDS,), jnp.float32),
            pltpu.VMEM((TBL_WORDS,), jnp.int32),
            pltpu.VMEM((NBINS,), jnp.float32),
            pltpu.VMEM_SHARED((B * NBINS,), jnp.float32),
            pltpu.VMEM_SHARED((B * NBINS,), jnp.float32),
            pltpu.VMEM_SHARED((B * NBINS,), jnp.float32),
            pltpu.VMEM_SHARED((B * NBINS,), jnp.float32),
            pltpu.SemaphoreType.DMA,
            pltpu.SemaphoreType.DMA,
        ],
    )
    return f(okey_flat, lkey_flat)


def _suffix_excl(X):
    ut = (lax.broadcasted_iota(jnp.int32, (128, 128), 0)
          > lax.broadcasted_iota(jnp.int32, (128, 128), 1)).astype(jnp.float32)
    within = jnp.dot(X, ut, preferred_element_type=jnp.float32)
    rows = jnp.sum(X, axis=1, keepdims=True)
    m8 = (lax.broadcasted_iota(jnp.int32, (8, 8), 1)
          > lax.broadcasted_iota(jnp.int32, (8, 8), 0)).astype(jnp.float32)
    above = jnp.dot(m8, rows, preferred_element_type=jnp.float32)
    return within + above


def _tcc_body(ocnt_ref, osum_ref, lpos_ref, lneg_ref, sc_ref, w1_ref, w2_ref, sc2_ref):
    red = lambda ref: jnp.sum(ref[...], axis=(0, 1))
    cnt = red(ocnt_ref)
    sm = red(osum_ref)
    lp = red(lpos_ref)
    ln = red(lneg_ref)

    srow = sc_ref[0]
    li = lax.broadcasted_iota(jnp.int32, (1, 128), 1)
    g = lambda k: jnp.sum(jnp.where(li == k, srow, 0.0))
    pos_sum, n_pos, num_neg = g(0), g(1), g(2)
    tis_cnt, tis_max, bce0 = g(3), g(4), g(5)
    tp, fn, fp, p = g(6), g(7), g(8), g(9)

    bi = lax.broadcasted_iota(jnp.int32, (8, 128), 0) * 128 + \
        lax.broadcasted_iota(jnp.int32, (8, 128), 1)
    is0 = (bi == 0).astype(jnp.float32)

    cnt = cnt - is0 * (jnp.float32(TOTAL) - num_neg)
    S = _suffix_excl(cnt)
    Ssum = _suffix_excl(sm)
    m = jnp.minimum(jnp.maximum(0.0, jnp.float32(K_ALL) - n_pos), num_neg)
    T = S + cnt
    mask = (S < m) & (m <= T)
    pick = lambda A: jnp.sum(jnp.where(mask, A, 0.0))
    S_t, cnt_t, sum_t, above_sum = pick(S), pick(cnt), pick(sm), pick(Ssum)
    lo_g = lax.bitcast_convert_type((bi + BIN_OFF) << 18, jnp.float32)
    hi_g = lax.bitcast_convert_type((bi + 1 + BIN_OFF) << 18, jnp.float32)
    lo, hi = pick(lo_g), pick(hi_g)
    mu = sum_t / jnp.maximum(cnt_t, 1e-30)
    h = jnp.maximum(0.0, jnp.minimum(hi - mu, mu - lo))
    kprime = m - S_t
    phi = kprime / jnp.maximum(cnt_t, 1e-30)
    neg_sum = above_sum + kprime * (mu + h * (1.0 - phi))
    kept = n_pos + m
    kept_loss = (pos_sum + neg_sum) / kept
    empty_loss = jnp.where(tis_cnt > 0, tis_max, bce0)
    ohem_i = jnp.where(kept == 0, empty_loss, kept_loss)

    tv = (tp + SMOOTH) / (tp + ALPHA * fn + BETA * fp + SMOOTH)
    omt = 1.0 - tv
    ft_i = jnp.where(omt > 0, jnp.exp(GAMMA * jnp.log(jnp.maximum(omt, 1e-38))), 0.0)

    CB = _suffix_excl(lp)
    NB = _suffix_excl(ln)
    n_neg_c = jnp.float32(TOTAL) - p
    a1 = p + NB + 0.5 * ln
    a2 = p + NB + 0.5 * (ln - 1.0)
    w1n = 1.0 / a1
    w2n = (p - CB - 0.5 * lp) / (a2 * (a2 + 1.0))
    w1a = (CB + 0.5 * (lp + 1.0)) / jnp.maximum(p, 1e-30)
    allpos = n_neg_c == 0
    w1_ref[0] = jnp.where(allpos, w1a, w1n)
    w2_ref[0] = jnp.where(allpos, 0.0, w2n)

    posb = jnp.where(p > 0, 1.0, 0.0)
    sc2_ref[0] = jnp.where(li == 0, ohem_i, 0.0) + \
        jnp.where(li == 1, ft_i, 0.0) + jnp.where(li == 2, posb, 0.0)


def _run_tcc(hists, scal):
    bs_h = pl.BlockSpec((2, 1, 8, 128), lambda s: (0, s, 0, 0))
    bs_s = pl.BlockSpec((1, 1, 128), lambda s: (s, 0, 0))
    bs_w = pl.BlockSpec((1, 8, 128), lambda s: (s, 0, 0))
    return pl.pallas_call(
        _tcc_body,
        grid=(B,),
        in_specs=[bs_h, bs_h, bs_h, bs_h, bs_s],
        out_specs=[bs_w, bs_w, bs_s],
        out_shape=[
            jax.ShapeDtypeStruct((B, 8, 128), jnp.float32),
            jax.ShapeDtypeStruct((B, 8, 128), jnp.float32),
            jax.ShapeDtypeStruct((B, 1, 128), jnp.float32),
        ],
    )(*hists, scal)


def _scd_body(w1f, w2f, lkey, lout, w1v, w2v, w1r, w2r, buf2, accv, sem0, sem1):
    core = lax.axis_index("c")
    sub = lax.axis_index("s")
    wid = sub * 2 + core
    lane = lax.iota(jnp.int32, 16)
    pltpu.sync_copy(w1f.at[pl.ds(sub * NBINS, NBINS)], w1v)
    pltpu.sync_copy(w2f.at[pl.ds(sub * NBINS, NBINS)], w2v)

    def rep(i, carry):
        a = w1v[pl.ds(i * 16, 16)]
        b = w2v[pl.ds(i * 16, 16)]
        bins16 = (jnp.zeros((16,), jnp.int32) + i * 16 + lane) * 16
        for l in range(16):
            plsc.store_scatter(w1r, [bins16 + l], a)
            plsc.store_scatter(w2r, [bins16 + l], b)
        return carry

    lax.fori_loop(0, NBINS // 16, rep, 0)

    def lov_batch(vs, acc):
        bits_l = [lax.bitcast_convert_type(v, jnp.int32) for v in vs]
        idxs = [_bin16(v)[1] * 16 + lane for v in vs]
        was = [plsc.load_gather(w1r, [idx]) for idx in idxs]
        wbs = [plsc.load_gather(w2r, [idx]) for idx in idxs]
        for v, b, wa, wb in zip(vs, bits_l, was, wbs):
            acc = acc + v * jnp.where((b & 1) == 1, wa, wb)
        return acc

    def sl(k):
        return lkey.at[sub, core * 32 + k // 2, pl.ds((k % 2) * SC_CHUNK, SC_CHUNK)]

    acc = _dbuf_stream(sl, buf2, sem0, sem1, lov_batch,
                       jnp.zeros((16,), jnp.float32))
    accv[pl.ds(0, 16)] = acc
    pltpu.sync_copy(accv, lout.at[pl.ds(wid * 16, 16)])


def _run_scd(w1, w2, lkey_flat):
    mesh = plsc.VectorSubcoreMesh(core_axis_name="c", subcore_axis_name="s")
    f = pl.kernel(
        _scd_body,
        mesh=mesh,
        compiler_params=pltpu.CompilerParams(needs_layout_passes=False),
        out_type=[jax.ShapeDtypeStruct((NTEC * 16,), jnp.float32)],
        scratch_types=[
            pltpu.VMEM((NBINS,), jnp.float32),
            pltpu.VMEM((NBINS,), jnp.float32),
            pltpu.VMEM((TBL_WORDS,), jnp.float32),
            pltpu.VMEM((TBL_WORDS,), jnp.float32),
            pltpu.VMEM((2 * SC_CHUNK,), jnp.float32),
            pltpu.VMEM((16,), jnp.float32),
            pltpu.SemaphoreType.DMA,
            pltpu.SemaphoreType.DMA,
        ],
    )
    return f(w1, w2, lkey_flat)


def _tce_body(sc2_ref, lov_ref, out_ref):
    sc2 = sc2_ref[...]
    li = lax.broadcasted_iota(jnp.int32, (B, 1, 128), 2)
    col = lambda k: jnp.sum(jnp.where(li == k, sc2, 0.0), axis=(1, 2))
    ohem_i, ft_i, posb = col(0), col(1), col(2)
    lov_i = jnp.sum(lov_ref[...], axis=1)
    n_pos_b = jnp.sum(posb)
    ohem_term = jnp.sum(ohem_i) / jnp.float32(B)
    ft_term = jnp.sum(jnp.where(posb > 0, ft_i, 0.0)) / n_pos_b
    lov_term = jnp.sum(jnp.where(posb > 0, lov_i, 0.0)) / n_pos_b
    full = ohem_term + ft_term + LOVASZ_WEIGHT * lov_term
    out_ref[...] = jnp.broadcast_to(jnp.where(n_pos_b > 0, full, ohem_term), (1, 1))


def _run_tce(sc2, lovpart):
    return pl.pallas_call(
        _tce_body,
        out_shape=jax.ShapeDtypeStruct((1, 1), jnp.float32),
    )(sc2, lovpart.reshape(B, NTEC * 16 // B))


def kernel(logits, targets, tissue_mask):
    x = logits.reshape(B, ROWS_A, LANES_A)
    tgt = targets.reshape(B, ROWS_A, LANES_A)
    tis = tissue_mask.reshape(B, ROWS_A, LANES_A)

    okey, lkey, scal = _run_tca(x, tgt, tis)

    hists = _run_scb(okey, lkey)
    hists4 = [h.reshape(2, B, 8, 128) for h in hists]

    w1, w2, sc2 = _run_tcc(hists4, scal)
    (lovpart,) = _run_scd(w1.reshape(-1), w2.reshape(-1), lkey)

    out = _run_tce(sc2, lovpart)
    return out.reshape(())
